# baseline (speedup 1.0000x reference)
import functools

import jax
import jax.numpy as jnp
from jax import lax
from jax.experimental import pallas as pl
from jax.experimental.pallas import tpu as pltpu
from jax.experimental.pallas import tpu_sc as plsc

B = 16384
V = 1000000
D = 32
NEG = 10

NCORES = 2
NSUB = 16
NW = NCORES * NSUB

CW = 512
U_PER_W = B // NW
N_PER_W = B * NEG // NW
NCHUNKS = N_PER_W // CW

TB = 512
NT = B // TB
RPT = TB * (1 + NEG)


def _sc_gather(ui, pi, nij, utT, itT, ntT):
    mesh = plsc.VectorSubcoreMesh(core_axis_name="c", subcore_axis_name="s")

    @functools.partial(
        pl.kernel,
        mesh=mesh,
        compiler_params=pltpu.CompilerParams(
            use_tc_tiling_on_sc=False, needs_layout_passes=False),
        out_type=[
            jax.ShapeDtypeStruct((D, B), jnp.float32),
            jax.ShapeDtypeStruct((D, B), jnp.float32),
            jax.ShapeDtypeStruct((D, NEG * B), jnp.float32),
        ],
        scratch_types=[
            pltpu.VMEM((N_PER_W,), jnp.int32),
            pltpu.VMEM((2, D, CW), jnp.float32),
            pltpu.SemaphoreType.DMA,
            pltpu.SemaphoreType.DMA,
            pltpu.SemaphoreType.DMA,
            pltpu.SemaphoreType.DMA,
        ],
    )
    def k(ui_h, pi_h, nij_h, utT_h, itT_h, ntT_h, uoT, poT, noT,
          idx_v, rows, ga, gb, sa, sb):
        wid = lax.axis_index("s") * NCORES + lax.axis_index("c")
        gsem = [ga, gb]
        ssem = [sa, sb]

        def fire_chunk(tabT_h, ibase, slot):
            def body(f, carry):
                pltpu.async_copy(
                    tabT_h.at[f].at[idx_v.at[pl.ds(ibase, CW)]],
                    rows.at[slot, f], gsem[slot])
                return carry
            lax.fori_loop(0, D, body, 0, unroll=False)

        def wait(sem, slot):
            pltpu.make_async_copy(uoT.at[:, pl.ds(0, CW)], rows.at[slot],
                                  sem).wait()

        def store_chunk(outT, col0, slot):
            pltpu.async_copy(rows.at[slot],
                             outT.at[:, pl.ds(col0, CW)], ssem[slot])

        pltpu.sync_copy(ui_h.at[pl.ds(wid * U_PER_W, U_PER_W)],
                        idx_v.at[pl.ds(0, U_PER_W)])
        pltpu.sync_copy(pi_h.at[pl.ds(wid * U_PER_W, U_PER_W)],
                        idx_v.at[pl.ds(U_PER_W, U_PER_W)])
        fire_chunk(utT_h, 0, 0)
        fire_chunk(itT_h, U_PER_W, 1)
        wait(gsem[0], 0)
        store_chunk(uoT, wid * U_PER_W, 0)
        wait(gsem[1], 1)
        store_chunk(poT, wid * U_PER_W, 1)
        pltpu.sync_copy(nij_h.at[pl.ds(wid * N_PER_W, N_PER_W)], idx_v)
        wait(ssem[0], 0)
        fire_chunk(ntT_h, 0, 0)
        wait(ssem[1], 1)
        fire_chunk(ntT_h, CW, 1)

        def body(g, carry):
            for s in range(2):
                c = 2 * g + s
                wait(gsem[s], s)
                store_chunk(noT, wid * N_PER_W + c * CW, s)

                @pl.when(g < NCHUNKS // 2 - 1)
                def _():
                    wait(ssem[s], s)
                    fire_chunk(ntT_h, (c + 2) * CW, s)
            return carry

        lax.fori_loop(0, NCHUNKS // 2, body, 0, unroll=False)
        wait(ssem[0], 0)
        wait(ssem[1], 1)

    return k(ui, pi, nij, utT, itT, ntT)


def _tc_body(u_ref, p_ref, n_ref, w1_ref, b1_ref, w2_ref, b2_ref,
             w3_ref, b3_ref, wdg_ref, wdm_ref, bd_ref, out_ref):
    u = u_ref[...]
    p = p_ref[...]
    n3 = n_ref[...]

    users = jnp.concatenate([u] * (1 + NEG), axis=1)
    items = jnp.concatenate([p] + [n3[:, j, :] for j in range(NEG)],
                            axis=1)

    x = jnp.concatenate([users, items], axis=0)
    h = jnp.maximum(jnp.dot(w1_ref[...], x, preferred_element_type=jnp.float32)
                    + b1_ref[...], 0.0)
    h = jnp.maximum(jnp.dot(w2_ref[...], h, preferred_element_type=jnp.float32)
                    + b2_ref[...], 0.0)
    h = jnp.maximum(jnp.dot(w3_ref[...], h, preferred_element_type=jnp.float32)
                    + b3_ref[...], 0.0)

    g = jax.nn.sigmoid(users * items)

    logit = (jnp.dot(wdg_ref[...], g, preferred_element_type=jnp.float32)
             + jnp.dot(wdm_ref[...], h, preferred_element_type=jnp.float32)
             + bd_ref[0, 0])
    out_ref[...] = logit


def _tc_mlp(user_T, pos_T, neg_T3, W1T, b1, W2T, b2, W3T, b3, wdg, wdm, bd):
    full = lambda shape: pl.BlockSpec(shape, lambda i: (0, 0))
    full3 = lambda shape: pl.BlockSpec(shape, lambda i: (0, 0, 0))
    return pl.pallas_call(
        _tc_body,
        grid=(NT,),
        in_specs=[
            pl.BlockSpec((D, TB), lambda i: (0, i)),
            pl.BlockSpec((D, TB), lambda i: (0, i)),
            pl.BlockSpec((D, NEG, TB), lambda i: (0, 0, i)),
            full((64, 2 * D)), full((64, 1)),
            full((16, 64)), full((16, 1)),
            full((8, 16)), full((8, 1)),
            full((1, D)), full((1, 8)), full((1, 1)),
        ],
        out_specs=pl.BlockSpec((1, RPT), lambda i: (0, i)),
        out_shape=jax.ShapeDtypeStruct((1, NT * RPT), jnp.float32),
    )(user_T, pos_T, neg_T3, W1T, b1, W2T, b2, W3T, b3, wdg, wdm, bd)


def kernel(user_inputs, pos_inputs, neg_inputs, user_table, item_table,
           neg_item_table, W1, b1, W2, b2, W3, b3, Wd, bd):
    ui = user_inputs.reshape(-1).astype(jnp.int32)
    pi = pos_inputs.reshape(-1).astype(jnp.int32)
    nij = neg_inputs.astype(jnp.int32).T.reshape(-1)

    user_T, pos_T, neg_T = _sc_gather(
        ui, pi, nij, user_table.T, item_table.T, neg_item_table.T)

    neg_T3 = neg_T.reshape(D, NEG, B)

    wdg = Wd[:D].reshape(1, D)
    wdm = Wd[D:].reshape(1, 8)
    out = _tc_mlp(user_T, pos_T, neg_T3,
                  W1.T, b1.reshape(64, 1), W2.T, b2.reshape(16, 1),
                  W3.T, b3.reshape(8, 1), wdg, wdm, bd.reshape(1, 1))

    o = out.reshape(NT, RPT)
    pos_log = o[:, :TB].reshape(B, 1)
    neg_log = o[:, TB:].reshape(NT, NEG, TB).transpose(0, 2, 1).reshape(B, NEG)
    return jnp.concatenate([pos_log, neg_log], axis=1)


# speedup vs baseline: 5.2219x; 5.2219x over previous
import functools

import jax
import jax.numpy as jnp
from jax import lax
from jax.experimental import pallas as pl
from jax.experimental.pallas import tpu as pltpu
from jax.experimental.pallas import tpu_sc as plsc

B = 16384
V = 1000000
D = 32
NEG = 10

NCORES = 2
NSUB = 16
NW = NCORES * NSUB

CHUNK = 128

UCHUNKS = B // CHUNK // (NW // 2)
NCHUNKS = B * NEG // CHUNK // NW

TB = 512
NT = B // TB
ROWS_PER_TILE = TB * (1 + NEG)


def _sc_gather(uq, pq, nq, user_table, item_table, neg_table):
    mesh = plsc.VectorSubcoreMesh(core_axis_name="c", subcore_axis_name="s")

    @functools.partial(
        pl.kernel,
        mesh=mesh,
        compiler_params=pltpu.CompilerParams(
            use_tc_tiling_on_sc=False, needs_layout_passes=False),
        out_type=[
            jax.ShapeDtypeStruct((B, D), jnp.float32),
            jax.ShapeDtypeStruct((B, D), jnp.float32),
            jax.ShapeDtypeStruct((B * NEG, D), jnp.float32),
        ],
        scratch_types=[
            pltpu.VMEM((NCHUNKS, CHUNK), jnp.int32),
            pltpu.VMEM((2, CHUNK, D), jnp.float32),
            pltpu.SemaphoreType.DMA,
            pltpu.SemaphoreType.DMA,
            pltpu.SemaphoreType.DMA,
            pltpu.SemaphoreType.DMA,
        ],
    )
    def k(uq_h, pq_h, nq_h, ut_h, it_h, nt_h, uout, pout, nout,
          qbuf, rows, ga, gb, sa, sb):
        wid = lax.axis_index("s") * NCORES + lax.axis_index("c")
        gsem = [ga, gb]
        ssem = [sa, sb]

        def section(tab_h, out_h, nchunks, obase):
            def fire(j, s):
                pltpu.async_copy(tab_h.at[qbuf.at[j]], rows.at[s], gsem[s])

            def store(j, s):
                pltpu.async_copy(
                    rows.at[s], out_h.at[pl.ds(obase + j * CHUNK, CHUNK)],
                    ssem[s])

            def wait(sem, s):
                pltpu.make_async_copy(out_h.at[pl.ds(obase, CHUNK)],
                                      rows.at[s], sem).wait()

            fire(0, 0)
            fire(1, 1)

            def body(g, carry):
                for s in range(2):
                    c = 2 * g + s
                    wait(gsem[s], s)
                    store(c, s)

                    @pl.when(g < nchunks // 2 - 1)
                    def _():
                        wait(ssem[s], s)
                        fire(c + 2, s)
                return carry

            lax.fori_loop(0, nchunks // 2, body, 0, unroll=False)
            wait(ssem[0], 0)
            wait(ssem[1], 1)

        @pl.when(wid < NW // 2)
        def _():
            pltpu.sync_copy(uq_h.at[pl.ds(wid * UCHUNKS, UCHUNKS)],
                            qbuf.at[pl.ds(0, UCHUNKS)])
            section(ut_h, uout, UCHUNKS, wid * UCHUNKS * CHUNK)

        @pl.when(wid >= NW // 2)
        def _():
            w = wid - NW // 2
            pltpu.sync_copy(pq_h.at[pl.ds(w * UCHUNKS, UCHUNKS)],
                            qbuf.at[pl.ds(0, UCHUNKS)])
            section(it_h, pout, UCHUNKS, w * UCHUNKS * CHUNK)

        pltpu.sync_copy(nq_h.at[pl.ds(wid * NCHUNKS, NCHUNKS)], qbuf)
        section(nt_h, nout, NCHUNKS, wid * NCHUNKS * CHUNK)

    return k(uq, pq, nq, user_table, item_table, neg_table)


def _tc_body(u_ref, p_ref, n_ref, w1_ref, b1_ref, w2_ref, b2_ref,
             w3_ref, b3_ref, wdg_ref, wdm_ref, bd_ref, out_ref):
    u = u_ref[...]
    p = p_ref[...]
    n = n_ref[...]
    ut = jnp.broadcast_to(u[:, None, :], (TB, NEG, D)).reshape(TB * NEG, D)

    users = jnp.concatenate([u, ut], axis=0)
    items = jnp.concatenate([p, n], axis=0)

    x = jnp.concatenate([users, items], axis=1)
    h = jnp.maximum(jnp.dot(x, w1_ref[...], preferred_element_type=jnp.float32)
                    + b1_ref[...], 0.0)
    h = jnp.maximum(jnp.dot(h, w2_ref[...], preferred_element_type=jnp.float32)
                    + b2_ref[...], 0.0)
    h = jnp.maximum(jnp.dot(h, w3_ref[...], preferred_element_type=jnp.float32)
                    + b3_ref[...], 0.0)

    g = jax.nn.sigmoid(users * items)

    logit = (jnp.sum(g * wdg_ref[...], axis=1, keepdims=True)
             + jnp.sum(h * wdm_ref[...], axis=1, keepdims=True)
             + bd_ref[0, 0])
    out_ref[...] = logit


def _tc_mlp(user_rows, pos_rows, neg_rows2, W1, b1, W2, b2, W3, b3,
            wdg, wdm, bd):
    full = lambda shape: pl.BlockSpec(shape, lambda i: (0, 0))
    return pl.pallas_call(
        _tc_body,
        grid=(NT,),
        in_specs=[
            pl.BlockSpec((TB, D), lambda i: (i, 0)),
            pl.BlockSpec((TB, D), lambda i: (i, 0)),
            pl.BlockSpec((TB * NEG, D), lambda i: (i, 0)),
            full((2 * D, 64)), full((1, 64)),
            full((64, 16)), full((1, 16)),
            full((16, 8)), full((1, 8)),
            full((1, D)), full((1, 8)), full((1, 1)),
        ],
        out_specs=pl.BlockSpec((ROWS_PER_TILE, 1), lambda i: (i, 0)),
        out_shape=jax.ShapeDtypeStruct((NT * ROWS_PER_TILE, 1), jnp.float32),
    )(user_rows, pos_rows, neg_rows2, W1, b1, W2, b2, W3, b3, wdg, wdm, bd)


def kernel(user_inputs, pos_inputs, neg_inputs, user_table, item_table,
           neg_item_table, W1, b1, W2, b2, W3, b3, Wd, bd):
    uq = user_inputs.reshape(B // CHUNK, CHUNK).astype(jnp.int32)
    pq = pos_inputs.reshape(B // CHUNK, CHUNK).astype(jnp.int32)
    nq = neg_inputs.reshape(B * NEG // CHUNK, CHUNK).astype(jnp.int32)

    user_rows, pos_rows, neg_rows = _sc_gather(
        uq, pq, nq, user_table, item_table, neg_item_table)

    wdg = Wd[:D].reshape(1, D)
    wdm = Wd[D:].reshape(1, 8)
    out = _tc_mlp(user_rows, pos_rows, neg_rows,
                  W1, b1.reshape(1, 64), W2, b2.reshape(1, 16),
                  W3, b3.reshape(1, 8), wdg, wdm, bd.reshape(1, 1))

    o = out.reshape(NT, ROWS_PER_TILE)
    pos_log = o[:, :TB].reshape(B, 1)
    neg_log = o[:, TB:].reshape(B, NEG)
    return jnp.concatenate([pos_log, neg_log], axis=1)
